# baseline (speedup 1.0000x reference)
import functools

import jax
import jax.numpy as jnp
from jax import lax
from jax.experimental import pallas as pl
from jax.experimental.pallas import tpu as pltpu
from jax.experimental.pallas import tpu_sc as plsc

_NUM_CORES = 2
_NUM_SUBCORES = 16


def kernel(tokens, pe):
    B, L = tokens.shape
    _, D = pe.shape

    rows_per_pair = B // _NUM_SUBCORES
    r_lo = 121
    r_hi = rows_per_pair - r_lo

    mesh = plsc.VectorSubcoreMesh(core_axis_name="c", subcore_axis_name="s")

    @functools.partial(
        pl.kernel,
        out_type=jax.ShapeDtypeStruct((B * L, D), jnp.float32),
        mesh=mesh,
        scratch_types=[
            pltpu.VMEM((L, D), jnp.float32),
            pltpu.SemaphoreType.DMA,
        ],
    )
    def pe_broadcast(pe_hbm, out_hbm, rep_v, sem):
        cid = lax.axis_index("c")
        sid = lax.axis_index("s")
        base = (sid * rows_per_pair + cid * r_lo) * L
        pltpu.sync_copy(pe_hbm.at[pl.ds(0, L)], rep_v)
        for i in range(r_lo):
            pltpu.make_async_copy(
                rep_v, out_hbm.at[pl.ds(base + i * L, L)], sem
            ).start()

        @pl.when(cid == 1)
        def _():
            for i in range(r_lo, r_hi):
                pltpu.make_async_copy(
                    rep_v, out_hbm.at[pl.ds(base + i * L, L)], sem
                ).start()

        for i in range(r_lo):
            pltpu.make_async_copy(
                rep_v, out_hbm.at[pl.ds(base + i * L, L)], sem
            ).wait()

        @pl.when(cid == 1)
        def _():
            for i in range(r_lo, r_hi):
                pltpu.make_async_copy(
                    rep_v, out_hbm.at[pl.ds(base + i * L, L)], sem
                ).wait()

    out = pe_broadcast(pe)
    return out.reshape(B, L, D)
